# Initial kernel scaffold; baseline (speedup 1.0000x reference)
import jax
import jax.numpy as jnp
from jax.experimental import pallas as pl


def kernel(x, num_tokens_per_expert, w1, w2, w3):
    raise NotImplementedError("write your pallas kernel here")


# speedup vs baseline: 4.0357x; 4.0357x over previous
import functools

import jax
import jax.numpy as jnp
from jax.experimental import pallas as pl
from jax.experimental.pallas import tpu as pltpu

_DIM = 2048
_HIDDEN = 4096
_E = 8
_TOKENS = 8192

_B = 256
_NB = _TOKENS // _B
_HC = 512
_K = _HIDDEN // _HC
_T = _NB + _E


def _metadata(counts):
    counts = counts.astype(jnp.int32)
    ends = jnp.cumsum(counts)
    starts = ends - counts
    row0 = jnp.arange(_NB, dtype=jnp.int32) * _B
    row1 = row0 + (_B - 1)
    e_lo = jnp.sum(row0[:, None] >= ends[None, :], axis=1).astype(jnp.int32)
    e_hi = jnp.sum(row1[:, None] >= ends[None, :], axis=1).astype(jnp.int32)
    e_hi = jnp.minimum(e_hi, _E - 1)
    n_b = jnp.where(e_lo <= _E - 1, e_hi - e_lo + 1, 1)
    off = jnp.cumsum(n_b)
    total = off[-1]
    t = jnp.arange(_T, dtype=jnp.int32)
    blk = jnp.sum(t[:, None] >= off[None, :], axis=1).astype(jnp.int32)
    blk = jnp.minimum(blk, _NB - 1)
    off0 = jnp.concatenate([jnp.zeros((1,), jnp.int32), off])
    slot = t - off0[blk]
    ex_raw = e_lo[blk] + slot
    valid = ((t < total) & (ex_raw <= _E - 1)).astype(jnp.int32)
    init = ((slot == 0) & (t < total)).astype(jnp.int32)
    ex = jnp.clip(ex_raw, 0, _E - 1)
    return ex, blk, valid, init, starts, ends


def _body(ex_s, blk_s, valid_s, init_s, starts_s, ends_s,
          x_ref, w1_ref, w2_ref, w3_ref, o_ref):
    t = pl.program_id(0)
    k = pl.program_id(1)

    @pl.when((k == 0) & (init_s[t] == 1))
    def _zero():
        o_ref[...] = jnp.zeros_like(o_ref)

    @pl.when(valid_s[t] == 1)
    def _compute():
        e = ex_s[t]
        rows = blk_s[t] * _B + jax.lax.broadcasted_iota(jnp.int32, (_B, 1), 0)
        mask = (rows >= starts_s[e]) & (rows < ends_s[e])
        xb = x_ref[...]
        dn = (((1,), (1,)), ((), ()))
        h1 = jax.lax.dot_general(xb, w1_ref[0], dn,
                                 preferred_element_type=jnp.float32)
        h3 = jax.lax.dot_general(xb, w3_ref[0], dn,
                                 preferred_element_type=jnp.float32)
        h = (h1 * jax.nn.sigmoid(h1)) * h3
        h = jnp.where(mask, h, 0.0).astype(jnp.bfloat16)
        o_ref[...] += jax.lax.dot_general(h, w2_ref[0], dn,
                                          preferred_element_type=jnp.float32)


@functools.partial(jax.jit, static_argnames=("interpret",))
def _grouped_mlp(x, counts, w1, w2, w3, interpret=False):
    ex, blk, valid, init, starts, ends = _metadata(counts)
    grid_spec = pltpu.PrefetchScalarGridSpec(
        num_scalar_prefetch=6,
        grid=(_T, _K),
        in_specs=[
            pl.BlockSpec((_B, _DIM),
                         lambda t, k, ex, blk, *_: (blk[t], 0)),
            pl.BlockSpec((1, _HC, _DIM),
                         lambda t, k, ex, blk, *_: (ex[t], k, 0)),
            pl.BlockSpec((1, _DIM, _HC),
                         lambda t, k, ex, blk, *_: (ex[t], 0, k)),
            pl.BlockSpec((1, _HC, _DIM),
                         lambda t, k, ex, blk, *_: (ex[t], k, 0)),
        ],
        out_specs=pl.BlockSpec((_B, _DIM),
                               lambda t, k, ex, blk, *_: (blk[t], 0)),
    )
    return pl.pallas_call(
        _body,
        grid_spec=grid_spec,
        out_shape=jax.ShapeDtypeStruct((_TOKENS, _DIM), jnp.float32),
        compiler_params=pltpu.CompilerParams(
            dimension_semantics=("arbitrary", "arbitrary")),
        interpret=interpret,
    )(ex, blk, valid, init, starts, ends,
      x.astype(jnp.bfloat16), w1.astype(jnp.bfloat16),
      w2.astype(jnp.bfloat16), w3.astype(jnp.bfloat16))


def kernel(x, num_tokens_per_expert, w1, w2, w3):
    return _grouped_mlp(x, num_tokens_per_expert, w1, w2, w3)


# speedup vs baseline: 4.7143x; 1.1681x over previous
import functools

import jax
import jax.numpy as jnp
from jax.experimental import pallas as pl
from jax.experimental.pallas import tpu as pltpu

_DIM = 2048
_HIDDEN = 4096
_E = 8
_TOKENS = 8192

_B = 256
_NB = _TOKENS // _B
_HC = 512
_K = _HIDDEN // _HC
_T = _NB + _E


def _metadata(counts):
    counts = counts.astype(jnp.int32)
    ends = jnp.cumsum(counts)
    starts = ends - counts
    row0 = jnp.arange(_NB, dtype=jnp.int32) * _B
    row1 = row0 + (_B - 1)
    e_lo = jnp.sum(row0[:, None] >= ends[None, :], axis=1).astype(jnp.int32)
    e_hi = jnp.sum(row1[:, None] >= ends[None, :], axis=1).astype(jnp.int32)
    e_hi = jnp.minimum(e_hi, _E - 1)
    n_b = jnp.where(e_lo <= _E - 1, e_hi - e_lo + 1, 1)
    off = jnp.cumsum(n_b)
    total = off[-1]
    t = jnp.arange(_T, dtype=jnp.int32)
    blk = jnp.sum(t[:, None] >= off[None, :], axis=1).astype(jnp.int32)
    blk = jnp.minimum(blk, _NB - 1)
    off0 = jnp.concatenate([jnp.zeros((1,), jnp.int32), off])
    slot = t - off0[blk]
    ex_raw = e_lo[blk] + slot
    valid = ((t < total) & (ex_raw <= _E - 1)).astype(jnp.int32)
    init = ((slot == 0) & (t < total)).astype(jnp.int32)
    ex = jnp.clip(ex_raw, 0, _E - 1)
    last_valid = jax.lax.cummax(jnp.where(valid == 1, t, -1))
    lv = jnp.maximum(last_valid, 0)
    exw = ex[lv]
    blkx = blk[lv]
    return ex, exw, blk, blkx, valid, init, starts, ends


def _body(ex_s, exw_s, blk_s, blkx_s, valid_s, init_s, starts_s, ends_s,
          x_ref, w1_ref, w2_ref, w3_ref, o_ref):
    t = pl.program_id(0)
    k = pl.program_id(1)

    @pl.when((k == 0) & (init_s[t] == 1))
    def _zero():
        o_ref[...] = jnp.zeros_like(o_ref)

    @pl.when(valid_s[t] == 1)
    def _compute():
        e = ex_s[t]
        rows = blk_s[t] * _B + jax.lax.broadcasted_iota(jnp.int32, (_B, 1), 0)
        mask = (rows >= starts_s[e]) & (rows < ends_s[e])
        xb = x_ref[...]
        dn = (((1,), (1,)), ((), ()))
        h1 = jax.lax.dot_general(xb, w1_ref[0], dn,
                                 preferred_element_type=jnp.float32)
        h3 = jax.lax.dot_general(xb, w3_ref[0], dn,
                                 preferred_element_type=jnp.float32)
        h = (h1 * jax.nn.sigmoid(h1)) * h3
        h = jnp.where(mask, h, 0.0).astype(jnp.bfloat16)
        o_ref[...] += jax.lax.dot_general(h, w2_ref[0], dn,
                                          preferred_element_type=jnp.float32)


@functools.partial(jax.jit, static_argnames=("interpret",))
def _grouped_mlp(x, counts, w1, w2, w3, interpret=False):
    ex, exw, blk, blkx, valid, init, starts, ends = _metadata(counts)
    grid_spec = pltpu.PrefetchScalarGridSpec(
        num_scalar_prefetch=8,
        grid=(_T, _K),
        in_specs=[
            pl.BlockSpec((_B, _DIM),
                         lambda t, k, ex, exw, blk, blkx, valid, *_:
                         (blkx[t], 0)),
            pl.BlockSpec((1, _HC, _DIM),
                         lambda t, k, ex, exw, blk, blkx, valid, *_:
                         (exw[t], k * valid[t], 0)),
            pl.BlockSpec((1, _DIM, _HC),
                         lambda t, k, ex, exw, blk, blkx, valid, *_:
                         (exw[t], 0, k * valid[t])),
            pl.BlockSpec((1, _HC, _DIM),
                         lambda t, k, ex, exw, blk, blkx, valid, *_:
                         (exw[t], k * valid[t], 0)),
        ],
        out_specs=pl.BlockSpec((_B, _DIM),
                               lambda t, k, ex, exw, blk, blkx, valid, *_:
                               (blk[t], 0)),
    )
    return pl.pallas_call(
        _body,
        grid_spec=grid_spec,
        out_shape=jax.ShapeDtypeStruct((_TOKENS, _DIM), jnp.float32),
        compiler_params=pltpu.CompilerParams(
            dimension_semantics=("arbitrary", "arbitrary")),
        interpret=interpret,
    )(ex, exw, blk, blkx, valid, init, starts, ends,
      x.astype(jnp.bfloat16), w1.astype(jnp.bfloat16),
      w2.astype(jnp.bfloat16), w3.astype(jnp.bfloat16))


def kernel(x, num_tokens_per_expert, w1, w2, w3):
    return _grouped_mlp(x, num_tokens_per_expert, w1, w2, w3)


# speedup vs baseline: 5.1102x; 1.0840x over previous
import functools

import jax
import jax.numpy as jnp
from jax.experimental import pallas as pl
from jax.experimental.pallas import tpu as pltpu

_DIM = 2048
_HIDDEN = 4096
_E = 8
_TOKENS = 8192

_B = 256
_NB = _TOKENS // _B
_HC = 512
_K = _HIDDEN // _HC
_J = 5
_GE = _E + 1
_NSTEP = _GE * _K * _J


def _metadata(counts):
    counts = counts.astype(jnp.int32)
    ends = jnp.cumsum(counts)
    starts = ends - counts
    nonempty = (counts > 0).astype(jnp.int32)
    sb = starts // _B
    eb = jnp.where(nonempty == 1, (ends - 1) // _B, sb)
    span = eb - sb + 1
    carried = jnp.concatenate(
        [jnp.zeros((1,), jnp.int32),
         (sb[1:] == eb[:-1]).astype(jnp.int32)])
    sb_next = jnp.concatenate([sb[1:], jnp.full((1,), _NB, jnp.int32)])
    nflush = jnp.minimum(span, jnp.maximum(sb_next - sb, 0))
    nflush = nflush.at[_E - 1].set(span[_E - 1])
    sticky = jnp.maximum(sb - 1, 0)
    ft = jnp.where(jnp.sum(counts) > 0, eb[_E - 1] + 1, 0)

    jj = jnp.arange(_J, dtype=jnp.int32)
    jvalid = ((jj[None, :] < span[:, None]) & (nonempty[:, None] == 1))
    zeroinit = ((jj[None, :] < span[:, None])
                & ~((jj[None, :] == 0) & (carried[:, None] == 1)))
    slot = (sb[:, None] + jj[None, :]) % _J
    jvalid = jvalid.astype(jnp.int32).reshape(-1)
    zeroinit = zeroinit.astype(jnp.int32).reshape(-1)
    slot = slot.reshape(-1)

    lastne = jax.lax.cummax(jnp.where(nonempty == 1,
                                      jnp.arange(_E, dtype=jnp.int32), -1))
    wex8 = jnp.maximum(lastne, 0)

    e3 = jnp.arange(_GE, dtype=jnp.int32).reshape(_GE, 1, 1)
    k3 = jnp.arange(_K, dtype=jnp.int32).reshape(1, _K, 1)
    j3 = jnp.arange(_J, dtype=jnp.int32).reshape(1, 1, _J)
    isreal = e3 < _E
    ec = jnp.minimum(e3, _E - 1)
    fetch = isreal & (nonempty[ec] == 1)
    widx_e = jnp.where(fetch, ec, wex8[ec])
    widx_k = jnp.where(fetch, k3, _K - 1)
    xidx = jnp.where(isreal,
                     jnp.minimum(sb[ec] + j3, eb[ec]),
                     eb[_E - 1])
    orow = jnp.where(nflush[:, None] > 0,
                     sb[:, None] + jnp.minimum(
                         jj[None, :], jnp.maximum(nflush[:, None] - 1, 0)),
                     sticky[:, None])
    oidx_real = jnp.where(k3 == _K - 1, orow[ec, j3], sticky[ec])
    oidx_tail = jnp.clip(ft + k3 * _J + j3, 0, _NB - 1)
    oidx = jnp.where(isreal, oidx_real, oidx_tail)

    widx_e = jnp.broadcast_to(widx_e, (_GE, _K, _J)).reshape(-1)
    widx_k = jnp.broadcast_to(widx_k, (_GE, _K, _J)).reshape(-1)
    xidx = jnp.broadcast_to(xidx, (_GE, _K, _J)).reshape(-1)
    oidx = jnp.broadcast_to(oidx, (_GE, _K, _J)).reshape(-1)

    ft = jnp.reshape(ft, (1,))
    return (widx_e, widx_k, xidx, oidx, nonempty, sb, starts, ends,
            nflush, jvalid, zeroinit, slot, ft)


def _fid(e, k, j):
    return (e * _K + k) * _J + j


def _body(widx_e, widx_k, xidx, oidx, nonempty_s, sb_s, starts_s, ends_s,
          nflush_s, jvalid_s, zeroinit_s, slot_s, ft_s,
          x_ref, w1_ref, w2_ref, w3_ref, o_ref,
          acc_ref, w1b_ref, w2b_ref, w3b_ref):
    e = pl.program_id(0)
    k = pl.program_id(1)
    j = pl.program_id(2)
    f = e * _J + j

    @pl.when(e < _E)
    def _real():
        @pl.when((j == 0) & (nonempty_s[e] == 1))
        def _cast():
            w1b_ref[...] = w1_ref[0].astype(jnp.bfloat16)
            w3b_ref[...] = w3_ref[0].astype(jnp.bfloat16)
            w2b_ref[...] = w2_ref[0].astype(jnp.bfloat16)

        @pl.when((k == 0) & (zeroinit_s[f] == 1))
        def _zero():
            s = slot_s[f]
            acc_ref[pl.ds(s * _B, _B), :] = jnp.zeros((_B, _DIM), jnp.float32)

        @pl.when(jvalid_s[f] == 1)
        def _compute():
            s = slot_s[f]
            b = sb_s[e] + j
            rows = b * _B + jax.lax.broadcasted_iota(jnp.int32, (_B, 1), 0)
            mask = (rows >= starts_s[e]) & (rows < ends_s[e])
            xb = x_ref[...]
            dn = (((1,), (1,)), ((), ()))
            h1 = jax.lax.dot_general(xb, w1b_ref[...], dn,
                                     preferred_element_type=jnp.float32)
            h3 = jax.lax.dot_general(xb, w3b_ref[...], dn,
                                     preferred_element_type=jnp.float32)
            h = (h1 * jax.nn.sigmoid(h1)) * h3
            h = jnp.where(mask, h, 0.0).astype(jnp.bfloat16)
            acc_ref[pl.ds(s * _B, _B), :] += jax.lax.dot_general(
                h, w2b_ref[...], dn, preferred_element_type=jnp.float32)

        @pl.when((k == _K - 1) & (j < nflush_s[e]))
        def _flush():
            s = slot_s[f]
            o_ref[...] = acc_ref[pl.ds(s * _B, _B), :]

    @pl.when(e == _E)
    def _tail():
        @pl.when(ft_s[0] + k * _J + j <= _NB - 1)
        def _zero_tail():
            o_ref[...] = jnp.zeros_like(o_ref)


@functools.partial(jax.jit, static_argnames=("interpret",))
def _grouped_mlp(x, counts, w1, w2, w3, interpret=False):
    meta = _metadata(counts)
    grid_spec = pltpu.PrefetchScalarGridSpec(
        num_scalar_prefetch=13,
        grid=(_GE, _K, _J),
        in_specs=[
            pl.BlockSpec((_B, _DIM),
                         lambda e, k, j, widx_e, widx_k, xidx, oidx, *_:
                         (xidx[_fid(e, k, j)], 0)),
            pl.BlockSpec((1, _HC, _DIM),
                         lambda e, k, j, widx_e, widx_k, xidx, oidx, *_:
                         (widx_e[_fid(e, k, j)], widx_k[_fid(e, k, j)], 0)),
            pl.BlockSpec((1, _DIM, _HC),
                         lambda e, k, j, widx_e, widx_k, xidx, oidx, *_:
                         (widx_e[_fid(e, k, j)], 0, widx_k[_fid(e, k, j)])),
            pl.BlockSpec((1, _HC, _DIM),
                         lambda e, k, j, widx_e, widx_k, xidx, oidx, *_:
                         (widx_e[_fid(e, k, j)], widx_k[_fid(e, k, j)], 0)),
        ],
        out_specs=pl.BlockSpec((_B, _DIM),
                               lambda e, k, j, widx_e, widx_k, xidx, oidx, *_:
                               (oidx[_fid(e, k, j)], 0)),
        scratch_shapes=[
            pltpu.VMEM((_J * _B, _DIM), jnp.float32),
            pltpu.VMEM((_HC, _DIM), jnp.bfloat16),
            pltpu.VMEM((_DIM, _HC), jnp.bfloat16),
            pltpu.VMEM((_HC, _DIM), jnp.bfloat16),
        ],
    )
    return pl.pallas_call(
        _body,
        grid_spec=grid_spec,
        out_shape=jax.ShapeDtypeStruct((_TOKENS, _DIM), jnp.float32),
        compiler_params=pltpu.CompilerParams(
            dimension_semantics=("arbitrary", "arbitrary", "arbitrary")),
        interpret=interpret,
    )(*meta, x.astype(jnp.bfloat16), w1, w2, w3)


def kernel(x, num_tokens_per_expert, w1, w2, w3):
    return _grouped_mlp(x, num_tokens_per_expert, w1, w2, w3)
